# baseline (speedup 1.0000x reference)
import functools

import jax
import jax.numpy as jnp
from jax import lax
from jax.experimental import pallas as pl
from jax.experimental.pallas import tpu as pltpu
from jax.experimental.pallas import tpu_sc as plsc

_NC = 2
_NS = 16
_SEQ = 2048
_D = 64
_KPW = 8
_JW = 1024
_ROWS_PER_CORE = _SEQ // _NC
_IB = 8
_NIB = _ROWS_PER_CORE // _IB
_NBUF = 4


def _gather_t(idx, table_t):
    mesh = plsc.VectorSubcoreMesh(core_axis_name="c", subcore_axis_name="s")

    @functools.partial(
        pl.kernel,
        out_type=jax.ShapeDtypeStruct((_SEQ, _D, _SEQ), jnp.float32),
        mesh=mesh,
        scratch_types=(
            [pltpu.VMEM((_KPW, 4097), jnp.float32)]
            + [pltpu.VMEM((_IB, _JW), jnp.int32)] * 2
            + [pltpu.VMEM((1, _KPW, _JW), jnp.float32)] * _NBUF
            + [pltpu.SemaphoreType.DMA] * (2 + _NBUF)
        ),
        compiler_params=pltpu.CompilerParams(needs_layout_passes=False),
    )
    def k(idx_hbm, tab_hbm, out_hbm, tabv, *bufs):
        idxv = bufs[0:2]
        outv = bufs[2:2 + _NBUF]
        sem_idx = bufs[2 + _NBUF:4 + _NBUF]
        sem_out = bufs[4 + _NBUF:4 + 2 * _NBUF]
        c = lax.axis_index("c")
        s = lax.axis_index("s")
        k0 = (s % 8) * _KPW
        j0 = (s // 8) * _JW
        row0 = c * _ROWS_PER_CORE

        pltpu.sync_copy(tab_hbm.at[pl.ds(k0, _KPW)], tabv)

        def issue_idx(g, b):
            pltpu.async_copy(
                idx_hbm.at[pl.ds(row0 + g * _IB, _IB), pl.ds(j0, _JW)],
                idxv[b], sem_idx[b])

        def wait_idx(b):
            pltpu.make_async_copy(
                idx_hbm.at[pl.ds(0, _IB), pl.ds(j0, _JW)],
                idxv[b], sem_idx[b]).wait()

        def issue_store(i, b):
            pltpu.async_copy(
                outv[b],
                out_hbm.at[pl.ds(row0 + i, 1), pl.ds(k0, _KPW), pl.ds(j0, _JW)],
                sem_out[b])

        def wait_store(b):
            pltpu.make_async_copy(
                outv[b],
                out_hbm.at[pl.ds(0, 1), pl.ds(k0, _KPW), pl.ds(j0, _JW)],
                sem_out[b]).wait()

        def compute(bi, r, bo):
            @plsc.parallel_loop(0, _JW // 16, unroll=4)
            def _(j16):
                iv = idxv[bi][r, pl.ds(j16 * 16, 16)]
                vals = [plsc.load_gather(
                            tabv, [jnp.full((16,), kr, jnp.int32), iv])
                        for kr in range(_KPW)]
                for kr in range(_KPW):
                    outv[bo][0, kr, pl.ds(j16 * 16, 16)] = vals[kr]

        issue_idx(0, 0)
        issue_idx(1, 1)

        def block(g, bi):
            wait_idx(bi)
            for r in range(_IB):
                i = g * _IB + r
                bo = r % _NBUF

                @pl.when(i >= _NBUF)
                def _():
                    wait_store(bo)

                compute(bi, r, bo)
                issue_store(i, bo)

            @pl.when(g + 2 < _NIB)
            def _():
                issue_idx(g + 2, bi)

        def outer(g2, carry):
            block(g2 * 2, 0)
            block(g2 * 2 + 1, 1)
            return carry

        lax.fori_loop(0, _NIB // 2, outer, 0)
        for b in range(_NBUF):
            wait_store(b)

    return k(idx, table_t)


def kernel(input, embeddings):
    table_t = jnp.swapaxes(embeddings, 0, 1)
    out = _gather_t(input.astype(jnp.int32), table_t)
    return jnp.swapaxes(out, 1, 2)
